# Initial kernel scaffold; baseline (speedup 1.0000x reference)
import jax
import jax.numpy as jnp
from jax.experimental import pallas as pl


def kernel(x, a, e):
    raise NotImplementedError("write your pallas kernel here")


# speedup vs baseline: 1.0165x; 1.0165x over previous
import jax
import jax.numpy as jnp
from jax.experimental import pallas as pl

_N = 10000
_D = 128
_TJ = 512


def _mm_kernel(e_ref, x_ref, o_ref):
    eb = e_ref[...].astype(jnp.bfloat16)
    xb = x_ref[...].astype(jnp.bfloat16)
    o_ref[...] = jax.lax.dot_general(
        eb, xb, (((0,), (0,)), ((), ())),
        preferred_element_type=jnp.float32,
    )


def kernel(x, a, e):
    x0 = x[0]
    return pl.pallas_call(
        _mm_kernel,
        grid=(pl.cdiv(_N, _TJ),),
        in_specs=[
            pl.BlockSpec((_N, _TJ), lambda j: (0, j)),
            pl.BlockSpec((_N, _D), lambda j: (0, 0)),
        ],
        out_specs=pl.BlockSpec((_TJ, _D), lambda j: (j, 0)),
        out_shape=jax.ShapeDtypeStruct((_N, _D), jnp.float32),
    )(e, x0)


# speedup vs baseline: 1.0171x; 1.0005x over previous
import jax
import jax.numpy as jnp
from jax.experimental import pallas as pl
from jax.experimental.pallas import tpu as pltpu

_N = 10000
_D = 128
_TJ = 512


def _mm_kernel(e_ref, x_ref, o_ref):
    eb = e_ref[...].astype(jnp.bfloat16)
    xb = x_ref[...].astype(jnp.bfloat16)
    o_ref[...] = jax.lax.dot_general(
        eb, xb, (((0,), (0,)), ((), ())),
        preferred_element_type=jnp.float32,
    )


def kernel(x, a, e):
    x0 = x[0]
    return pl.pallas_call(
        _mm_kernel,
        grid=(pl.cdiv(_N, _TJ),),
        in_specs=[
            pl.BlockSpec((_N, _TJ), lambda j: (0, j)),
            pl.BlockSpec((_N, _D), lambda j: (0, 0)),
        ],
        out_specs=pl.BlockSpec((_TJ, _D), lambda j: (j, 0)),
        out_shape=jax.ShapeDtypeStruct((_N, _D), jnp.float32),
        compiler_params=pltpu.CompilerParams(
            dimension_semantics=("parallel",),
        ),
    )(e, x0)
